# baseline (speedup 1.0000x reference)
import jax
import jax.numpy as jnp
from jax.experimental import pallas as pl

N = 64
OBS_DIM = 1024
LOWD = 512
K = 512
H = 8
A = 128
HID = 512


def _fused(obs_ref, w_obs_ref, b_obs_ref, w1_ref, b1c_ref, idx_ref,
           w2_ref, b2c_ref, action_ref, value_ref):
    iota_a = jax.lax.broadcasted_iota(jnp.int32, (K, A), 1)
    actT = jnp.zeros((HID, K), dtype=jnp.float32)
    for h in range(H):
        onehot = (idx_ref[:, h:h + 1] == iota_a).astype(jnp.float32)
        actT = actT + jax.lax.dot_general(
            w1_ref[LOWD + h * A:LOWD + (h + 1) * A, :], onehot,
            (((0,), (1,)), ((), ())), preferred_element_type=jnp.float32)

    obs_lowd = jnp.dot(obs_ref[...], w_obs_ref[...],
                       preferred_element_type=jnp.float32) + b_obs_ref[...]
    oT = jax.lax.dot_general(
        w1_ref[:LOWD, :], obs_lowd, (((0,), (1,)), ((), ())),
        preferred_element_type=jnp.float32) + b1c_ref[...]
    w2 = w2_ref[...]
    slabs = []
    for b in range(N):
        tb = jnp.maximum(actT + oT[:, b:b + 1], 0.0)
        slabs.append(jax.lax.dot_general(
            w2, tb, (((0,), (0,)), ((), ())),
            preferred_element_type=jnp.float32))
    out3 = jnp.stack(slabs, axis=1) + b2c_ref[...][:, :, None]
    vals = out3[:H]
    lg = out3[H:]
    m = jnp.max(lg, axis=0, keepdims=True)
    e = jnp.exp(lg - m)
    s = jnp.sum(e, axis=0)
    v = jnp.sum(vals * e, axis=0) / s
    vmax = jnp.max(v, axis=1, keepdims=True)
    iota_k = jax.lax.broadcasted_iota(jnp.int32, (N, K), 1)
    karg = jnp.min(jnp.where(v >= vmax, iota_k, K), axis=1, keepdims=True)
    onehot_karg = (iota_k == karg).astype(jnp.float32)
    idx0f = idx_ref[:, 0:1].astype(jnp.float32)
    aidx = jax.lax.dot_general(
        onehot_karg, idx0f, (((1,), (0,)), ((), ())),
        preferred_element_type=jnp.float32).astype(jnp.int32)
    iota_act = jax.lax.broadcasted_iota(jnp.int32, (N, A), 1)
    action_ref[...] = (iota_act == aidx).astype(jnp.float32)
    value_ref[...] = vmax


@jax.jit
def kernel(observations, action_indices, W_obs, b_obs, W1, b1, W2, b2):
    idx = action_indices.reshape(K, H)
    action, value = pl.pallas_call(
        _fused,
        out_shape=(
            jax.ShapeDtypeStruct((N, A), jnp.float32),
            jax.ShapeDtypeStruct((N, 1), jnp.float32),
        ),
    )(observations, W_obs, b_obs.reshape(1, LOWD), W1, b1.reshape(HID, 1),
      idx, W2, b2.reshape(2 * H, 1))
    return (action, value.reshape(N))
